# Initial kernel scaffold; baseline (speedup 1.0000x reference)
import jax
import jax.numpy as jnp
from jax.experimental import pallas as pl


def kernel(inputs, embedding):
    raise NotImplementedError("write your pallas kernel here")


# speedup vs baseline: 1.2265x; 1.2265x over previous
import jax
import jax.numpy as jnp
from jax.experimental import pallas as pl
from jax.experimental.pallas import tpu as pltpu

_H = 4
_K = 1024
_C = 256
_N = 8192
_BN = 512
_NB = _N // _BN
_COMMIT = 0.25


def _vq_body(x_ref, emb_ref, enc_ref, q_ref, loss_ref, perp_ref,
             counts_ref, esq_ref):
    i = pl.program_id(0)

    @pl.when(i == 0)
    def _init():
        counts_ref[...] = jnp.zeros_like(counts_ref)
        loss_ref[...] = jnp.zeros_like(loss_ref)
        perp_ref[...] = jnp.zeros_like(perp_ref)
        emb = emb_ref[...]
        esq_ref[...] = jnp.sum(emb * emb, axis=2)

    loss_part = jnp.zeros((1, 1), jnp.float32)
    for h in range(_H):
        xh = x_ref[:, h * _C:(h + 1) * _C]
        embh = emb_ref[h]
        xh_bf = xh.astype(jnp.bfloat16)
        dots = jax.lax.dot_general(
            xh_bf, embh, (((1,), (1,)), ((), ())),
            preferred_element_type=jnp.float32)
        xsq = jnp.sum(xh * xh, axis=1, keepdims=True)
        dist = (xsq + esq_ref[pl.ds(h, 1), :]) - 2.0 * dots
        idx = jnp.argmin(dist, axis=1, keepdims=True)
        lane = jax.lax.broadcasted_iota(jnp.int32, (_BN, _K), 1)
        oh = (lane == idx)
        oh_f = oh.astype(jnp.float32)
        enc_ref[:, h * _K:(h + 1) * _K] = oh_f
        counts_ref[...] += jnp.sum(oh_f.reshape(_BN // 8, 8, _K), axis=0)
        oh_bf = oh.astype(jnp.bfloat16)
        qh = jax.lax.dot_general(
            oh_bf, embh, (((1,), (0,)), ((), ())),
            preferred_element_type=jnp.float32)
        q_ref[:, h * _C:(h + 1) * _C] = xh + (qh - xh)
        diff = qh - xh
        loss_part = loss_part + jnp.sum(diff * diff).reshape(1, 1)

    loss_ref[...] += loss_part

    @pl.when(i == _NB - 1)
    def _fin():
        total = jnp.sum(counts_ref[...], axis=0, keepdims=True)
        avg = total * (1.0 / (_N * _H))
        ent = jnp.sum(avg * jnp.log(avg + 1e-10), axis=1, keepdims=True)
        perp_ref[...] = jnp.exp(-ent)
        loss_ref[...] = loss_ref[...] * (_COMMIT / (_N * _H * _C))


def kernel(inputs, embedding):
    B, T = inputs.shape[0], inputs.shape[1]
    x2d = inputs.reshape(_N, _H * _C)
    enc2d, q2d, loss, perp = pl.pallas_call(
        _vq_body,
        grid=(_NB,),
        in_specs=[
            pl.BlockSpec((_BN, _H * _C), lambda i: (i, 0)),
            pl.BlockSpec((_H, _K, _C), lambda i: (0, 0, 0)),
        ],
        out_specs=[
            pl.BlockSpec((_BN, _H * _K), lambda i: (i, 0)),
            pl.BlockSpec((_BN, _H * _C), lambda i: (i, 0)),
            pl.BlockSpec((1, 1), lambda i: (0, 0)),
            pl.BlockSpec((1, 1), lambda i: (0, 0)),
        ],
        out_shape=[
            jax.ShapeDtypeStruct((_N, _H * _K), jnp.float32),
            jax.ShapeDtypeStruct((_N, _H * _C), jnp.float32),
            jax.ShapeDtypeStruct((1, 1), jnp.float32),
            jax.ShapeDtypeStruct((1, 1), jnp.float32),
        ],
        scratch_shapes=[
            pltpu.VMEM((8, _K), jnp.float32),
            pltpu.VMEM((_H, _K), jnp.float32),
        ],
        compiler_params=pltpu.CompilerParams(
            dimension_semantics=("arbitrary",),
            vmem_limit_bytes=60 * 1024 * 1024,
        ),
    )(x2d, embedding)
    quantized = q2d.reshape(B, T, _H, _C)
    enc = enc2d.reshape(B, T, _H, _K)
    return (loss.reshape(()), quantized, perp.reshape(()), enc)


# speedup vs baseline: 2.8388x; 2.3146x over previous
import jax
import jax.numpy as jnp
from jax.experimental import pallas as pl
from jax.experimental.pallas import tpu as pltpu

_H = 4
_K = 1024
_C = 256
_N = 8192
_BN = 512
_NB = _N // _BN
_COMMIT = 0.25


def _vq_body(x_ref, emb_ref, enc_ref, q_ref, loss_ref, perp_ref,
             counts_ref, esq_ref):
    i = pl.program_id(0)

    @pl.when(i == 0)
    def _init():
        counts_ref[...] = jnp.zeros_like(counts_ref)
        loss_ref[...] = jnp.zeros_like(loss_ref)
        perp_ref[...] = jnp.zeros_like(perp_ref)
        emb = emb_ref[...]
        esq_ref[...] = jnp.sum(emb * emb, axis=2)

    loss_part = jnp.zeros((1, 1), jnp.float32)
    for h in range(_H):
        xh = x_ref[:, h * _C:(h + 1) * _C]
        embh = emb_ref[h]
        xh_bf = xh.astype(jnp.bfloat16)
        dots = jax.lax.dot_general(
            xh_bf, embh, (((1,), (1,)), ((), ())),
            preferred_element_type=jnp.float32)
        xsq = jnp.sum(xh * xh, axis=1, keepdims=True)
        dist = (xsq + esq_ref[pl.ds(h, 1), :]) - 2.0 * dots
        idx = jnp.argmin(dist, axis=1, keepdims=True)
        lane = jax.lax.broadcasted_iota(jnp.int32, (_BN, _K), 1)
        oh = (lane == idx)
        oh_f = oh.astype(jnp.float32)
        enc_ref[:, h, :] = oh_f
        counts_ref[...] += jnp.sum(oh_f.reshape(_BN // 8, 8, _K), axis=0)
        oh_bf = oh.astype(jnp.bfloat16)
        qh = jax.lax.dot_general(
            oh_bf, embh, (((1,), (0,)), ((), ())),
            preferred_element_type=jnp.float32)
        q_ref[:, h, :] = xh + (qh - xh)
        diff = qh - xh
        loss_part = loss_part + jnp.sum(diff * diff).reshape(1, 1)

    loss_ref[...] += loss_part

    @pl.when(i == _NB - 1)
    def _fin():
        total = jnp.sum(counts_ref[...], axis=0, keepdims=True)
        avg = total * (1.0 / (_N * _H))
        ent = jnp.sum(avg * jnp.log(avg + 1e-10), axis=1, keepdims=True)
        perp_ref[...] = jnp.exp(-ent)
        loss_ref[...] = loss_ref[...] * (_COMMIT / (_N * _H * _C))


def kernel(inputs, embedding):
    B, T = inputs.shape[0], inputs.shape[1]
    x2d = inputs.reshape(_N, _H * _C)
    enc2d, q2d, loss, perp = pl.pallas_call(
        _vq_body,
        grid=(_NB,),
        in_specs=[
            pl.BlockSpec((_BN, _H * _C), lambda i: (i, 0)),
            pl.BlockSpec((_H, _K, _C), lambda i: (0, 0, 0)),
        ],
        out_specs=[
            pl.BlockSpec((_BN, _H, _K), lambda i: (i, 0, 0)),
            pl.BlockSpec((_BN, _H, _C), lambda i: (i, 0, 0)),
            pl.BlockSpec((1, 1), lambda i: (0, 0)),
            pl.BlockSpec((1, 1), lambda i: (0, 0)),
        ],
        out_shape=[
            jax.ShapeDtypeStruct((_N, _H, _K), jnp.float32),
            jax.ShapeDtypeStruct((_N, _H, _C), jnp.float32),
            jax.ShapeDtypeStruct((1, 1), jnp.float32),
            jax.ShapeDtypeStruct((1, 1), jnp.float32),
        ],
        scratch_shapes=[
            pltpu.VMEM((8, _K), jnp.float32),
            pltpu.VMEM((_H, _K), jnp.float32),
        ],
        compiler_params=pltpu.CompilerParams(
            dimension_semantics=("arbitrary",),
            vmem_limit_bytes=60 * 1024 * 1024,
        ),
    )(x2d, embedding)
    quantized = q2d.reshape(B, T, _H, _C)
    enc = enc2d.reshape(B, T, _H, _K)

    return (loss.reshape(()), quantized, perp.reshape(()), enc)


# speedup vs baseline: 2.9539x; 1.0405x over previous
import jax
import jax.numpy as jnp
from jax.experimental import pallas as pl
from jax.experimental.pallas import tpu as pltpu

_H = 4
_K = 1024
_C = 256
_N = 8192
_BN = 512
_NB = _N // _BN
_COMMIT = 0.25


def _vq_body(x_ref, emb_ref, enc_ref, q_ref, loss_ref, perp_ref,
             counts_ref, esq_ref):
    i = pl.program_id(0)

    @pl.when(i == 0)
    def _init():
        counts_ref[...] = jnp.zeros_like(counts_ref)
        loss_ref[...] = jnp.zeros_like(loss_ref)
        perp_ref[...] = jnp.zeros_like(perp_ref)
        emb = emb_ref[...]
        esq_ref[...] = jnp.sum(emb * emb, axis=2)

    loss_part = jnp.zeros((1, 1), jnp.float32)
    for h in range(_H):
        xh = x_ref[:, h * _C:(h + 1) * _C]
        embh = emb_ref[h]
        xh_bf2 = (xh_bf := xh.astype(jnp.bfloat16)) + xh_bf
        dots2 = jax.lax.dot_general(
            xh_bf2, embh, (((1,), (1,)), ((), ())),
            preferred_element_type=jnp.float32)
        xsq = jnp.sum(xh * xh, axis=1, keepdims=True)
        dist = (xsq + esq_ref[pl.ds(h, 1), :]) - dots2
        idx = jnp.argmin(dist, axis=1, keepdims=True)
        lane = jax.lax.broadcasted_iota(jnp.int32, (_BN, _K), 1)
        oh = (lane == idx)
        oh_f = jnp.where(oh, jnp.float32(1.0), jnp.float32(0.0))
        enc_ref[:, h, :] = oh_f
        counts_ref[...] += jnp.sum(oh_f.reshape(_BN // 8, 8, _K), axis=0)
        oh_bf = oh.astype(jnp.bfloat16)
        qh = jax.lax.dot_general(
            oh_bf, embh, (((1,), (0,)), ((), ())),
            preferred_element_type=jnp.float32)
        q_ref[:, h, :] = xh + (qh - xh)
        diff = qh - xh
        loss_part = loss_part + jnp.sum(diff * diff).reshape(1, 1)

    loss_ref[...] += loss_part

    @pl.when(i == _NB - 1)
    def _fin():
        total = jnp.sum(counts_ref[...], axis=0, keepdims=True)
        avg = total * (1.0 / (_N * _H))
        ent = jnp.sum(avg * jnp.log(avg + 1e-10), axis=1, keepdims=True)
        perp_ref[...] = jnp.exp(-ent)
        loss_ref[...] = loss_ref[...] * (_COMMIT / (_N * _H * _C))


def kernel(inputs, embedding):
    B, T = inputs.shape[0], inputs.shape[1]
    x2d = inputs.reshape(_N, _H * _C)
    enc2d, q2d, loss, perp = pl.pallas_call(
        _vq_body,
        grid=(_NB,),
        in_specs=[
            pl.BlockSpec((_BN, _H * _C), lambda i: (i, 0)),
            pl.BlockSpec((_H, _K, _C), lambda i: (0, 0, 0)),
        ],
        out_specs=[
            pl.BlockSpec((_BN, _H, _K), lambda i: (i, 0, 0)),
            pl.BlockSpec((_BN, _H, _C), lambda i: (i, 0, 0)),
            pl.BlockSpec((1, 1), lambda i: (0, 0)),
            pl.BlockSpec((1, 1), lambda i: (0, 0)),
        ],
        out_shape=[
            jax.ShapeDtypeStruct((_N, _H, _K), jnp.float32),
            jax.ShapeDtypeStruct((_N, _H, _C), jnp.float32),
            jax.ShapeDtypeStruct((1, 1), jnp.float32),
            jax.ShapeDtypeStruct((1, 1), jnp.float32),
        ],
        scratch_shapes=[
            pltpu.VMEM((8, _K), jnp.float32),
            pltpu.VMEM((_H, _K), jnp.float32),
        ],
        compiler_params=pltpu.CompilerParams(
            dimension_semantics=("arbitrary",),
            vmem_limit_bytes=60 * 1024 * 1024,
        ),
    )(x2d, embedding)
    quantized = q2d.reshape(B, T, _H, _C)
    enc = enc2d.reshape(B, T, _H, _K)

    return (loss.reshape(()), quantized, perp.reshape(()), enc)


# speedup vs baseline: 3.0395x; 1.0290x over previous
import jax
import jax.numpy as jnp
from jax.experimental import pallas as pl
from jax.experimental.pallas import tpu as pltpu

_H = 4
_K = 1024
_C = 256
_N = 8192
_BN = 1024
_NB = _N // _BN
_COMMIT = 0.25


def _vq_body(x_ref, emb_ref, enc_ref, q_ref, loss_ref, perp_ref,
             counts_ref, esq_ref):
    i = pl.program_id(0)

    @pl.when(i == 0)
    def _init():
        counts_ref[...] = jnp.zeros_like(counts_ref)
        loss_ref[...] = jnp.zeros_like(loss_ref)
        perp_ref[...] = jnp.zeros_like(perp_ref)
        emb = emb_ref[...]
        esq_ref[...] = jnp.sum(emb * emb, axis=2)

    loss_part = jnp.zeros((1, 1), jnp.float32)
    for h in range(_H):
        xh = x_ref[:, h * _C:(h + 1) * _C]
        embh = emb_ref[h]
        xh_bf2 = (xh_bf := xh.astype(jnp.bfloat16)) + xh_bf
        dots2 = jax.lax.dot_general(
            xh_bf2, embh, (((1,), (1,)), ((), ())),
            preferred_element_type=jnp.float32)
        xsq = jnp.sum(xh * xh, axis=1, keepdims=True)
        dist = (xsq + esq_ref[pl.ds(h, 1), :]) - dots2
        idx = jnp.argmin(dist, axis=1, keepdims=True)
        lane = jax.lax.broadcasted_iota(jnp.int32, (_BN, _K), 1)
        oh = (lane == idx)
        oh_f = jnp.where(oh, jnp.float32(1.0), jnp.float32(0.0))
        enc_ref[:, h, :] = oh_f
        counts_ref[...] += jnp.sum(oh_f.reshape(_BN // 8, 8, _K), axis=0)
        oh_bf = oh.astype(jnp.bfloat16)
        qh = jax.lax.dot_general(
            oh_bf, embh, (((1,), (0,)), ((), ())),
            preferred_element_type=jnp.float32)
        q_ref[:, h, :] = xh + (qh - xh)
        diff = qh - xh
        loss_part = loss_part + jnp.sum(diff * diff).reshape(1, 1)

    loss_ref[...] += loss_part

    @pl.when(i == _NB - 1)
    def _fin():
        total = jnp.sum(counts_ref[...], axis=0, keepdims=True)
        avg = total * (1.0 / (_N * _H))
        ent = jnp.sum(avg * jnp.log(avg + 1e-10), axis=1, keepdims=True)
        perp_ref[...] = jnp.exp(-ent)
        loss_ref[...] = loss_ref[...] * (_COMMIT / (_N * _H * _C))


def kernel(inputs, embedding):
    B, T = inputs.shape[0], inputs.shape[1]
    x2d = inputs.reshape(_N, _H * _C)
    enc2d, q2d, loss, perp = pl.pallas_call(
        _vq_body,
        grid=(_NB,),
        in_specs=[
            pl.BlockSpec((_BN, _H * _C), lambda i: (i, 0)),
            pl.BlockSpec((_H, _K, _C), lambda i: (0, 0, 0)),
        ],
        out_specs=[
            pl.BlockSpec((_BN, _H, _K), lambda i: (i, 0, 0)),
            pl.BlockSpec((_BN, _H, _C), lambda i: (i, 0, 0)),
            pl.BlockSpec((1, 1), lambda i: (0, 0)),
            pl.BlockSpec((1, 1), lambda i: (0, 0)),
        ],
        out_shape=[
            jax.ShapeDtypeStruct((_N, _H, _K), jnp.float32),
            jax.ShapeDtypeStruct((_N, _H, _C), jnp.float32),
            jax.ShapeDtypeStruct((1, 1), jnp.float32),
            jax.ShapeDtypeStruct((1, 1), jnp.float32),
        ],
        scratch_shapes=[
            pltpu.VMEM((8, _K), jnp.float32),
            pltpu.VMEM((_H, _K), jnp.float32),
        ],
        compiler_params=pltpu.CompilerParams(
            dimension_semantics=("arbitrary",),
            vmem_limit_bytes=60 * 1024 * 1024,
        ),
    )(x2d, embedding)
    quantized = q2d.reshape(B, T, _H, _C)
    enc = enc2d.reshape(B, T, _H, _K)

    return (loss.reshape(()), quantized, perp.reshape(()), enc)
